# Initial kernel scaffold; baseline (speedup 1.0000x reference)
import jax
import jax.numpy as jnp
from jax.experimental import pallas as pl


def kernel(sem_feat_company, sem_feat_brand, sem_feat_organize, W0, W1, y):
    raise NotImplementedError("write your pallas kernel here")


# speedup vs baseline: 3.4584x; 3.4584x over previous
import jax
import jax.numpy as jnp
from jax.experimental import pallas as pl

_NC, _NB, _NO = 63180, 34588, 4148
_N = _NC + _NB + _NO
_D = 128
_EPS = 1e-8
_BLK = 2048


def _means_body(x_ref, y_ref, sums_ref, cnt_ref):
    i = pl.program_id(0)
    x = x_ref[...]
    y = y_ref[...]
    rows = i * x.shape[0] + jax.lax.broadcasted_iota(
        jnp.int32, (x.shape[0], 1), 0)
    valid = rows < _NC
    yv = jnp.where(valid, y, 0.0)
    xv = jnp.where(valid, x, 0.0)
    ts = jnp.sum(xv * yv, axis=0)
    sa = jnp.sum(xv, axis=0)
    c = jnp.sum(yv)

    @pl.when(i == 0)
    def _init():
        sums_ref[...] = jnp.zeros_like(sums_ref)
        cnt_ref[...] = jnp.zeros_like(cnt_ref)

    sums_ref[...] += jnp.concatenate([ts[None, :], sa[None, :]], axis=0)
    cnt_ref[...] += c


def _cos_core(x, w0, w1, tm, fm):
    wm = 0.5 * (w0 + w1)
    s2 = wm * wm
    sx = s2 * x
    na2 = jnp.sum(sx * x, axis=1)
    dt = jnp.sum(sx * tm, axis=1)
    df = jnp.sum(sx * fm, axis=1)
    nt2 = jnp.sum(s2 * (tm * tm), axis=1)
    nf2 = jnp.sum(s2 * (fm * fm), axis=1)
    na = jnp.maximum(jnp.sqrt(na2), _EPS)
    nt = jnp.maximum(jnp.sqrt(nt2), _EPS)
    nf = jnp.maximum(jnp.sqrt(nf2), _EPS)
    t = dt / (na * nt)
    f = df / (na * nf)
    return t, f


def _cos_body(x_ref, w0_ref, w1_ref, tm_ref, fm_ref, sem_ref, pred_ref):
    t, f = _cos_core(x_ref[...], w0_ref[...], w1_ref[...],
                     tm_ref[...], fm_ref[...])
    sem_ref[...] = jnp.stack([t, f], axis=0)
    pred_ref[...] = (f > t).astype(jnp.int32)[None, :]


def _cos_ce_body(x_ref, w0_ref, w1_ref, tm_ref, fm_ref, y_ref,
                 sem_ref, pred_ref, loss_ref):
    i = pl.program_id(0)
    t, f = _cos_core(x_ref[...], w0_ref[...], w1_ref[...],
                     tm_ref[...], fm_ref[...])
    sem_ref[...] = jnp.stack([t, f], axis=0)
    pred_ref[...] = (f > t).astype(jnp.int32)[None, :]
    m = jnp.maximum(t, f)
    lse = m + jnp.log(jnp.exp(t - m) + jnp.exp(f - m))
    yv = y_ref[...][:, 0]
    chosen = t + yv * (f - t)
    rows = i * t.shape[0] + jax.lax.broadcasted_iota(
        jnp.int32, t.shape, 0)
    contrib = jnp.where(rows < _NC, lse - chosen, 0.0)

    @pl.when(i == 0)
    def _init():
        loss_ref[...] = jnp.zeros_like(loss_ref)

    loss_ref[...] += jnp.sum(contrib)


def _cos_call(body, x, w0, w1, tm, fm, n_rows, extra=()):
    n_extra = len(extra)
    g = pl.cdiv(n_rows, _BLK)
    in_specs = [
        pl.BlockSpec((_BLK, _D), lambda i: (i, 0)),
        pl.BlockSpec((_BLK, _D), lambda i: (i, 0)),
        pl.BlockSpec((_BLK, _D), lambda i: (i, 0)),
        pl.BlockSpec((1, _D), lambda i: (0, 0)),
        pl.BlockSpec((1, _D), lambda i: (0, 0)),
    ] + [pl.BlockSpec((_BLK, 1), lambda i: (i, 0))] * n_extra
    out_specs = [
        pl.BlockSpec((2, _BLK), lambda i: (0, i)),
        pl.BlockSpec((1, _BLK), lambda i: (0, i)),
    ]
    out_shape = [
        jax.ShapeDtypeStruct((2, n_rows), jnp.float32),
        jax.ShapeDtypeStruct((1, n_rows), jnp.int32),
    ]
    if n_extra:
        out_specs.append(pl.BlockSpec((1, 1), lambda i: (0, 0)))
        out_shape.append(jax.ShapeDtypeStruct((1, 1), jnp.float32))
    return pl.pallas_call(
        body, grid=(g,), in_specs=in_specs,
        out_specs=out_specs, out_shape=out_shape,
    )(x, w0, w1, tm, fm, *extra)


def kernel(sem_feat_company, sem_feat_brand, sem_feat_organize, W0, W1, y):
    y_f = y.astype(jnp.float32).reshape(_NC, 1)

    bm = 4096
    sums, cnt = pl.pallas_call(
        _means_body,
        grid=(pl.cdiv(_NC, bm),),
        in_specs=[pl.BlockSpec((bm, _D), lambda i: (i, 0)),
                  pl.BlockSpec((bm, 1), lambda i: (i, 0))],
        out_specs=[pl.BlockSpec((2, _D), lambda i: (0, 0)),
                   pl.BlockSpec((1, 1), lambda i: (0, 0))],
        out_shape=[jax.ShapeDtypeStruct((2, _D), jnp.float32),
                   jax.ShapeDtypeStruct((1, 1), jnp.float32)],
    )(sem_feat_company, y_f)

    tcnt = cnt[0, 0]
    tmean = (sums[0] / jnp.maximum(tcnt, 1.0)).reshape(1, _D)
    fmean = ((sums[1] - sums[0]) / jnp.maximum(_NC - tcnt, 1.0)).reshape(1, _D)

    sem_c, pred_c, loss = _cos_call(
        _cos_ce_body, sem_feat_company, W0, W1, tmean, fmean,
        n_rows=_NC, extra=(y_f,))

    sem_b, pred_b = _cos_call(
        _cos_body, sem_feat_brand, W0[_NC:_NC + _NB], W1[_NC:_NC + _NB],
        tmean, fmean, n_rows=_NB)

    sem_o, pred_o = _cos_call(
        _cos_body, sem_feat_organize, W0[_NC + _NB:], W1[_NC + _NB:],
        tmean, fmean, n_rows=_NO)

    semantic = jnp.concatenate([sem_c, sem_b, sem_o], axis=1)
    pseudo_loss = loss[0, 0] / _NC
    return (semantic, pseudo_loss,
            pred_c[0], pred_b[0], pred_o[0])
